# Initial kernel scaffold; baseline (speedup 1.0000x reference)
import jax
import jax.numpy as jnp
from jax.experimental import pallas as pl


def kernel(X1, X2):
    raise NotImplementedError("write your pallas kernel here")


# speedup vs baseline: 2.2010x; 2.2010x over previous
import functools

import jax
import jax.numpy as jnp
from jax import lax
from jax.experimental import pallas as pl
from jax.experimental.pallas import tpu as pltpu
from jax.experimental.pallas import tpu_sc as plsc

L = 4
N = 2 * L + 1
NF = N * N

_PAIRS = [[(m1, mu + L - m1) for m1 in range(max(0, mu - L), min(N, mu + L + 1))]
          for mu in range(N)]
_ALL_PAIRS = [pq for mup in range(N) for pq in _PAIRS[mup]]

B_IN = 20000
NUM_WORKERS = 32
ROWS_PER_WORKER = 640
CHUNK = 160
NCHUNKS = ROWS_PER_WORKER // CHUNK
GROUPS = CHUNK // 16
B_PAD = NUM_WORKERS * ROWS_PER_WORKER


def _body(x1_hbm, x2_hbm, out_hbm, x1_v, x2_v, out_v):
    nc = 2
    wid = lax.axis_index("s") * nc + lax.axis_index("c")
    base = wid * (ROWS_PER_WORKER * NF)
    lane81 = lax.broadcasted_iota(jnp.int32, (16,), 0) * NF

    def chunk_body(ci, carry):
        cb = base + ci * (CHUNK * NF)
        pltpu.sync_copy(x1_hbm.at[pl.ds(cb, CHUNK * NF)], x1_v)
        pltpu.sync_copy(x2_hbm.at[pl.ds(cb, CHUNK * NF)], x2_v)

        def group_body(g, c2):
            row81 = g * (16 * NF) + lane81
            for mu in range(N):
                acc = [None] * N
                for (m1, m2) in _PAIRS[mu]:
                    a = [plsc.load_gather(x1_v, [row81 + (m1 * N + j)])
                         for j in range(N)]
                    b = [plsc.load_gather(x2_v, [row81 + (m2 * N + j)])
                         for j in range(N)]
                    for (m1p, m2p) in _ALL_PAIRS:
                        mup = m1p + m2p - L
                        t = a[m1p] * b[m2p]
                        acc[mup] = t if acc[mup] is None else acc[mup] + t
                for mup in range(N):
                    plsc.store_scatter(out_v, [row81 + (mu * N + mup)],
                                       acc[mup])
            return c2

        lax.fori_loop(0, GROUPS, group_body, 0)
        pltpu.sync_copy(out_v, out_hbm.at[pl.ds(cb, CHUNK * NF)])
        return carry

    lax.fori_loop(0, NCHUNKS, chunk_body, 0)


@jax.jit
def kernel(X1, X2):
    b = X1.shape[0]
    x1f = X1.reshape(b * NF)
    x2f = X2.reshape(b * NF)
    pad = (B_PAD - b) * NF
    x1f = jnp.pad(x1f, (0, pad))
    x2f = jnp.pad(x2f, (0, pad))

    mesh = plsc.VectorSubcoreMesh(core_axis_name="c", subcore_axis_name="s")
    run = pl.kernel(
        _body,
        out_type=jax.ShapeDtypeStruct((B_PAD * NF,), jnp.float32),
        mesh=mesh,
        compiler_params=pltpu.CompilerParams(needs_layout_passes=False),
        scratch_types=[
            pltpu.VMEM((CHUNK * NF,), jnp.float32),
            pltpu.VMEM((CHUNK * NF,), jnp.float32),
            pltpu.VMEM((CHUNK * NF,), jnp.float32),
        ],
    )
    out = run(x1f, x2f)
    return out[:b * NF].reshape(b, N, N)


# speedup vs baseline: 2.2121x; 1.0051x over previous
import jax
import jax.numpy as jnp
from jax import lax
from jax.experimental import pallas as pl
from jax.experimental.pallas import tpu as pltpu
from jax.experimental.pallas import tpu_sc as plsc

L = 4
N = 2 * L + 1
NF = N * N

_PAIRS = [[(m1, mu + L - m1) for m1 in range(max(0, mu - L), min(N, mu + L + 1))]
          for mu in range(N)]

B_IN = 20000
NUM_WORKERS = 32
ROWS_PER_WORKER = 640
CHUNK = 160
NCHUNKS = ROWS_PER_WORKER // CHUNK
GROUPS = CHUNK // 16


def _tree_sum(vals):
    while len(vals) > 1:
        nxt = [a + b for a, b in zip(vals[::2], vals[1::2])]
        if len(vals) % 2:
            nxt.append(vals[-1])
        vals = nxt
    return vals[0]


def _body(x1_hbm, x2_hbm, out_hbm, x1_v, x2_v, out_v):
    nc = 2
    wid = lax.axis_index("s") * nc + lax.axis_index("c")
    base = wid * ROWS_PER_WORKER
    lane81 = lax.broadcasted_iota(jnp.int32, (16,), 0) * NF

    def chunk_body(ci, carry):
        cb = jnp.minimum(base + ci * CHUNK, B_IN - CHUNK) * NF
        pltpu.sync_copy(x1_hbm.at[pl.ds(cb, CHUNK * NF)], x1_v)
        pltpu.sync_copy(x2_hbm.at[pl.ds(cb, CHUNK * NF)], x2_v)

        def group_body(g, c2):
            row81 = g * (16 * NF) + lane81
            for mu in range(N):
                acc = [None] * N
                for (m1, m2) in _PAIRS[mu]:
                    a = [plsc.load_gather(x1_v, [row81 + (m1 * N + j)])
                         for j in range(N)]
                    b = [plsc.load_gather(x2_v, [row81 + (m2 * N + j)])
                         for j in range(N)]
                    for mup in range(N):
                        part = _tree_sum([a[p] * b[q] for (p, q) in _PAIRS[mup]])
                        acc[mup] = part if acc[mup] is None else acc[mup] + part
                for mup in range(N):
                    plsc.store_scatter(out_v, [row81 + (mu * N + mup)],
                                       acc[mup])
            return c2

        lax.fori_loop(0, GROUPS, group_body, 0)
        pltpu.sync_copy(out_v, out_hbm.at[pl.ds(cb, CHUNK * NF)])
        return carry

    lax.fori_loop(0, NCHUNKS, chunk_body, 0)


@jax.jit
def kernel(X1, X2):
    b = X1.shape[0]
    x1f = X1.reshape(b * NF)
    x2f = X2.reshape(b * NF)

    mesh = plsc.VectorSubcoreMesh(core_axis_name="c", subcore_axis_name="s")
    run = pl.kernel(
        _body,
        out_type=jax.ShapeDtypeStruct((b * NF,), jnp.float32),
        mesh=mesh,
        compiler_params=pltpu.CompilerParams(needs_layout_passes=False),
        scratch_types=[
            pltpu.VMEM((CHUNK * NF,), jnp.float32),
            pltpu.VMEM((CHUNK * NF,), jnp.float32),
            pltpu.VMEM((CHUNK * NF,), jnp.float32),
        ],
    )
    out = run(x1f, x2f)
    return out.reshape(b, N, N)
